# Initial kernel scaffold; baseline (speedup 1.0000x reference)
import jax
import jax.numpy as jnp
from jax.experimental import pallas as pl


def kernel(x, edge_index, batch, Wl1, bl1, Wr1, Wl2, bl2, Wr2, Wl3, bl3, Wr3, Wc, bc, W1, b1, W2, b2):
    raise NotImplementedError("write your pallas kernel here")


# speedup vs baseline: 4.5904x; 4.5904x over previous
import functools

import jax
import jax.numpy as jnp
from jax import lax
from jax.experimental import pallas as pl
from jax.experimental.pallas import tpu as pltpu
from jax.experimental.pallas import tpu_sc as plsc

N = 10000
E = 320000
H = 128
B = 64
K = 30
C = 10
CONV_OUT = 32
KW = 5
NT = K - KW + 1

NC = 2
NS = 16
NW = NC * NS
EPW = E // NW
CH = 80
NCHUNK = EPW // CH
SROWS = 624
SROWS_LAST = N - (NS - 1) * SROWS
ZR = 16
KP = 32
GPW = B // NW
NKC = N // 16

def _mesh():
    return plsc.VectorSubcoreMesh(core_axis_name="c", subcore_axis_name="s",
                                  num_cores=NC, num_subcores=NS)


def _make_sc_agg(W, interpret=False):
    @functools.partial(
        pl.kernel,
        out_type=jax.ShapeDtypeStruct((NC * N, W), jnp.float32),
        mesh=_mesh(),
        compiler_params=pltpu.CompilerParams(needs_layout_passes=False),
        scratch_types=[
            pltpu.VMEM((CH,), jnp.int32),
            pltpu.VMEM((CH,), jnp.int32),
            pltpu.VMEM((CH, W), jnp.float32),
            pltpu.VMEM((ZR, W), jnp.float32),
            pltpu.VMEM_SHARED((N, W), jnp.float32),
            pltpu.SemaphoreType.DMA,
        ],
        interpret=interpret,
    )
    def agg_kernel(table_hbm, src_hbm, dst_hbm, out_hbm,
                   sidx, didx, rows, zbuf, acc, sem):
        c = lax.axis_index("c")
        s = lax.axis_index("s")
        wid = c * NS + s

        def zrow(r, carry):
            for l in range(W // 16):
                zbuf[r, pl.ds(l * 16, 16)] = jnp.zeros((16,), jnp.float32)
            return carry
        lax.fori_loop(0, ZR, zrow, 0)
        nz = jnp.where(s == NS - 1, SROWS_LAST // ZR, SROWS // ZR)

        def zslice(j, carry):
            off = pl.multiple_of(s * SROWS + j * ZR, 8)
            pltpu.sync_copy(zbuf, acc.at[pl.ds(off, ZR)])
            return carry
        lax.fori_loop(0, nz, zslice, 0)
        plsc.subcore_barrier()

        base0 = wid * EPW

        def body(j, carry):
            base = pl.multiple_of(base0 + j * CH, 8)
            pltpu.sync_copy(src_hbm.at[pl.ds(base, CH)], sidx)
            pltpu.sync_copy(dst_hbm.at[pl.ds(base, CH)], didx)
            pltpu.async_copy(table_hbm.at[sidx], rows, sem).wait()
            pltpu.sync_copy(rows, acc.at[didx], add=True)
            return carry
        lax.fori_loop(0, NCHUNK, body, 0)
        plsc.subcore_barrier()

        off = pl.multiple_of(s * SROWS, 8)
        ooff = pl.multiple_of(c * N + s * SROWS, 8)

        @pl.when(s < NS - 1)
        def _wb():
            pltpu.sync_copy(acc.at[pl.ds(off, SROWS)],
                            out_hbm.at[pl.ds(ooff, SROWS)])

        @pl.when(s == NS - 1)
        def _wb_last():
            pltpu.sync_copy(acc.at[pl.ds(off, SROWS_LAST)],
                            out_hbm.at[pl.ds(ooff, SROWS_LAST)])

    return agg_kernel


def _make_sc_deg(interpret=False):
    @functools.partial(
        pl.kernel,
        out_type=jax.ShapeDtypeStruct((NW * N,), jnp.float32),
        mesh=_mesh(),
        compiler_params=pltpu.CompilerParams(needs_layout_passes=False),
        scratch_types=[
            pltpu.VMEM((EPW,), jnp.int32),
            pltpu.VMEM((N,), jnp.float32),
        ],
        interpret=interpret,
    )
    def deg_kernel(dst_hbm, out_hbm, didx, dloc):
        c = lax.axis_index("c")
        s = lax.axis_index("s")
        wid = c * NS + s

        def z(i, carry):
            dloc[pl.ds(i * 16, 16)] = jnp.zeros((16,), jnp.float32)
            return carry
        lax.fori_loop(0, N // 16, z, 0)

        pltpu.sync_copy(
            dst_hbm.at[pl.ds(pl.multiple_of(wid * EPW, 8), EPW)], didx)
        ones = jnp.ones((16,), jnp.float32)
        lanes = lax.iota(jnp.int32, 16)

        def body(i, carry):
            idx = didx[pl.ds(i * 16, 16)]
            for l in range(16):
                plsc.addupdate_scatter(dloc, [idx], ones, mask=lanes == l)
            return carry
        lax.fori_loop(0, EPW // 16, body, 0)

        pltpu.sync_copy(dloc, out_hbm.at[pl.ds(
            pl.multiple_of(wid * N, 8), N)])

    return deg_kernel


def _make_sc_pool(interpret=False):
    @functools.partial(
        pl.kernel,
        out_type=jax.ShapeDtypeStruct((B * KP, H), jnp.float32),
        mesh=_mesh(),
        compiler_params=pltpu.CompilerParams(needs_layout_passes=False),
        scratch_types=[
            pltpu.VMEM((N + 16,), jnp.float32),
            pltpu.VMEM((N,), jnp.int32),
            pltpu.VMEM((K,), jnp.int32),
            pltpu.VMEM((KP, H), jnp.float32),
            pltpu.SemaphoreType.DMA,
        ],
        interpret=interpret,
    )
    def pool_kernel(h_hbm, keys_hbm, batch_hbm, out_hbm,
                    keys_v, batch_v, sel_v, rows_v, sem):
        c = lax.axis_index("c")
        s = lax.axis_index("s")
        wid = c * NS + s
        pltpu.sync_copy(keys_hbm, keys_v.at[pl.ds(0, N)])
        pltpu.sync_copy(batch_hbm, batch_v)

        for ki in range(K, KP):
            for l in range(H // 16):
                rows_v[ki, pl.ds(l * 16, 16)] = jnp.zeros((16,), jnp.float32)

        for gi in range(GPW):
            g = wid * GPW + gi

            def cbody(i, carry):
                st, ct = carry
                bv = batch_v[pl.ds(i * 16, 16)]
                st = st + jnp.sum(jnp.where(bv < g, 1, 0))
                ct = ct + jnp.sum(jnp.where(bv == g, 1, 0))
                return st, ct
            start, cnt = lax.fori_loop(
                0, NKC, cbody, (jnp.int32(0), jnp.int32(0)))

            c0 = start // 16
            c1 = (start + cnt + 15) // 16

            def select(ki, carry):
                def scan(ci, sc_carry):
                    bk, bp = sc_carry
                    off = ci * 16
                    kv = keys_v[pl.ds(off, 16)]
                    pos = off + lax.iota(jnp.int32, 16)
                    valid = (pos >= start) & (pos < start + cnt)
                    kv = jnp.where(valid, kv, -1.0)
                    take = kv > bk
                    return (jnp.where(take, kv, bk),
                            jnp.where(take, pos, bp))
                bk, bp = lax.fori_loop(
                    c0, c1, scan,
                    (jnp.full((16,), -1.0, jnp.float32),
                     jnp.full((16,), N, jnp.int32)))
                m = jnp.max(bk)
                p = jnp.min(jnp.where(bk >= m, bp, N))
                plsc.store_scatter(
                    keys_v, [jnp.full((16,), p, jnp.int32)],
                    jnp.full((16,), -1.0, jnp.float32))
                plsc.store_scatter(
                    sel_v, [jnp.full((16,), ki, jnp.int32)],
                    jnp.full((16,), jnp.minimum(p, N - 1), jnp.int32))
                return carry
            lax.fori_loop(0, K, select, 0)

            pltpu.async_copy(h_hbm.at[sel_v], rows_v.at[pl.ds(0, K)],
                             sem).wait()

            for ki in range(K):
                @pl.when(cnt <= ki)
                def _zero(ki=ki):
                    for l in range(H // 16):
                        rows_v[ki, pl.ds(l * 16, 16)] = (
                            jnp.zeros((16,), jnp.float32))

            pltpu.sync_copy(rows_v, out_hbm.at[pl.ds(
                pl.multiple_of(g * KP, 8), KP)])

    return pool_kernel


def _tc_layer(a0, a1, degt, h, Wl, bl2, Wr, interpret=False):
    R = 1000

    def body(a0_r, a1_r, d_r, h_r, wl_r, bl_r, wr_r, o_r):
        dsum = jnp.sum(d_r[...], axis=1, keepdims=True)
        rdeg = 1.0 / jnp.maximum(dsum, 1.0)
        agg = (a0_r[...] + a1_r[...]) * rdeg
        o_r[...] = jnp.maximum(
            jnp.dot(agg, wl_r[...], preferred_element_type=jnp.float32)
            + jnp.dot(h_r[...], wr_r[...], preferred_element_type=jnp.float32)
            + bl_r[...], 0.0)

    return pl.pallas_call(
        body,
        grid=(N // R,),
        in_specs=[
            pl.BlockSpec((R, H), lambda i: (i, 0)),
            pl.BlockSpec((R, H), lambda i: (i, 0)),
            pl.BlockSpec((R, NW), lambda i: (i, 0)),
            pl.BlockSpec((R, H), lambda i: (i, 0)),
            pl.BlockSpec((H, H), lambda i: (0, 0)),
            pl.BlockSpec((1, H), lambda i: (0, 0)),
            pl.BlockSpec((H, H), lambda i: (0, 0)),
        ],
        out_specs=pl.BlockSpec((R, H), lambda i: (i, 0)),
        out_shape=jax.ShapeDtypeStruct((N, H), jnp.float32),
        interpret=interpret,
    )(a0, a1, degt, h, Wl, bl2, Wr)


def _tc_head(pooledT, Wck2, bc2, W1e, b12, W2, b22, interpret=False):
    def body(x0, x1, x2, x3, x4, wck_r, bc_r, w1e_r, b1_r, w2_r, b2_r,
             out_r, z1_s):
        i = pl.program_id(0)
        xs = (x0, x1, x2, x3, x4)
        ct = jnp.zeros((B, CONV_OUT), jnp.float32)
        for kw in range(KW):
            ct = ct + jnp.dot(xs[kw][...], wck_r[pl.ds(kw * H, H), :],
                              preferred_element_type=jnp.float32)
        crelu = jnp.maximum(ct + bc_r[...], 0.0)
        contrib = jnp.dot(crelu, w1e_r[...].reshape(CONV_OUT, H),
                          preferred_element_type=jnp.float32)

        @pl.when(i == 0)
        def _init():
            z1_s[...] = contrib

        @pl.when(i > 0)
        def _acc():
            z1_s[...] = z1_s[...] + contrib

        @pl.when(i == NT - 1)
        def _finish():
            z1 = jnp.maximum(z1_s[...] + b1_r[...], 0.0)
            z2 = jnp.dot(z1, w2_r[...],
                         preferred_element_type=jnp.float32) + b2_r[...]
            m = jnp.max(z2, axis=1, keepdims=True)
            lse = jnp.log(jnp.sum(jnp.exp(z2 - m), axis=1, keepdims=True))
            out_r[...] = z2 - m - lse

    return pl.pallas_call(
        body,
        grid=(NT,),
        in_specs=[
            pl.BlockSpec((B, H), lambda i: (i, 0)),
            pl.BlockSpec((B, H), lambda i: (i + 1, 0)),
            pl.BlockSpec((B, H), lambda i: (i + 2, 0)),
            pl.BlockSpec((B, H), lambda i: (i + 3, 0)),
            pl.BlockSpec((B, H), lambda i: (i + 4, 0)),
            pl.BlockSpec((KW * H, CONV_OUT), lambda i: (0, 0)),
            pl.BlockSpec((1, CONV_OUT), lambda i: (0, 0)),
            pl.BlockSpec((1, CONV_OUT, H), lambda i: (i, 0, 0)),
            pl.BlockSpec((1, H), lambda i: (0, 0)),
            pl.BlockSpec((H, C), lambda i: (0, 0)),
            pl.BlockSpec((1, C), lambda i: (0, 0)),
        ],
        out_specs=pl.BlockSpec((B, C), lambda i: (0, 0)),
        out_shape=jax.ShapeDtypeStruct((B, C), jnp.float32),
        scratch_shapes=[pltpu.VMEM((B, H), jnp.float32)],
        interpret=interpret,
    )(pooledT, pooledT, pooledT, pooledT, pooledT,
      Wck2, bc2, W1e, b12, W2, b22)


_sc_agg = None
_sc_deg = None
_sc_pool = None


def _get_sc_kernels():
    global _sc_agg, _sc_deg, _sc_pool
    if _sc_agg is None:
        _sc_agg = _make_sc_agg(H)
        _sc_deg = _make_sc_deg()
        _sc_pool = _make_sc_pool()
    return _sc_agg, _sc_deg, _sc_pool


def kernel(x, edge_index, batch, Wl1, bl1, Wr1, Wl2, bl2, Wr2,
           Wl3, bl3, Wr3, Wc, bc, W1, b1, W2, b2):
    agg, deg, pool = _get_sc_kernels()
    src = edge_index[0]
    dst = edge_index[1]

    degt = deg(dst).reshape(NW, N).T

    p = agg(x, src, dst)
    h = _tc_layer(p[:N], p[N:], degt, x, Wl1, bl1[None, :], Wr1)
    p = agg(h, src, dst)
    h = _tc_layer(p[:N], p[N:], degt, h, Wl2, bl2[None, :], Wr2)
    p = agg(h, src, dst)
    h = _tc_layer(p[:N], p[N:], degt, h, Wl3, bl3[None, :], Wr3)

    keys = h[:, H - 1]
    pooled = pool(h, keys, batch)

    pooledT = (pooled.reshape(B, KP, H)[:, :K, :]
               .transpose(1, 0, 2).reshape(K * B, H))
    Wck2 = Wc.transpose(2, 1, 0).reshape(KW * H, CONV_OUT)
    W1e = W1.reshape(CONV_OUT, NT, H).transpose(1, 0, 2)
    return _tc_head(pooledT, Wck2, bc[None, :], W1e, b1[None, :],
                    W2, b2[None, :])


# speedup vs baseline: 7.0444x; 1.5346x over previous
import functools

import jax
import jax.numpy as jnp
from jax import lax
from jax.experimental import pallas as pl
from jax.experimental.pallas import tpu as pltpu
from jax.experimental.pallas import tpu_sc as plsc

N = 10000
E = 320000
H = 128
B = 64
K = 30
C = 10
CONV_OUT = 32
KW = 5
NT = K - KW + 1

NC = 2
NS = 16
NW = NC * NS
EPW = E // NW
CH = 80
NCHUNK = EPW // CH
SROWS = 624
SROWS_LAST = N - (NS - 1) * SROWS
ZR = 16
KP = 32
GPW = B // NW
NKC = N // 16

def _mesh():
    return plsc.VectorSubcoreMesh(core_axis_name="c", subcore_axis_name="s",
                                  num_cores=NC, num_subcores=NS)


def _make_sc_agg(W, interpret=False):
    @functools.partial(
        pl.kernel,
        out_type=jax.ShapeDtypeStruct((NC * N, W), jnp.float32),
        mesh=_mesh(),
        compiler_params=pltpu.CompilerParams(needs_layout_passes=False),
        scratch_types=[
            pltpu.VMEM((CH,), jnp.int32),
            pltpu.VMEM((CH,), jnp.int32),
            pltpu.VMEM((CH,), jnp.int32),
            pltpu.VMEM((CH,), jnp.int32),
            pltpu.VMEM((CH, W), jnp.float32),
            pltpu.VMEM((CH, W), jnp.float32),
            pltpu.VMEM((ZR, W), jnp.float32),
            pltpu.VMEM_SHARED((N, W), jnp.float32),
            pltpu.SemaphoreType.DMA,
            pltpu.SemaphoreType.DMA,
        ],
        interpret=interpret,
    )
    def agg_kernel(table_hbm, src_hbm, dst_hbm, out_hbm,
                   sidx0, sidx1, didx0, didx1, rows0, rows1, zbuf, acc,
                   sem0, sem1):
        c = lax.axis_index("c")
        s = lax.axis_index("s")
        wid = c * NS + s

        def zrow(r, carry):
            for l in range(W // 16):
                zbuf[r, pl.ds(l * 16, 16)] = jnp.zeros((16,), jnp.float32)
            return carry
        lax.fori_loop(0, ZR, zrow, 0)
        nz = jnp.where(s == NS - 1, SROWS_LAST // ZR, SROWS // ZR)

        def zslice(j, carry):
            off = pl.multiple_of(s * SROWS + j * ZR, 8)
            pltpu.sync_copy(zbuf, acc.at[pl.ds(off, ZR)])
            return carry
        lax.fori_loop(0, nz, zslice, 0)
        plsc.subcore_barrier()

        base0 = wid * EPW
        bufs = (rows0, rows1)
        sbufs = (sidx0, sidx1)
        dbufs = (didx0, didx1)
        sems = (sem0, sem1)

        def _gather(j, b):
            base = pl.multiple_of(base0 + j * CH, 8)
            pltpu.sync_copy(src_hbm.at[pl.ds(base, CH)], sbufs[b])
            pltpu.sync_copy(dst_hbm.at[pl.ds(base, CH)], dbufs[b])
            pltpu.async_copy(table_hbm.at[sbufs[b]], bufs[b], sems[b])

        def _drain(b):
            pltpu.make_async_copy(
                table_hbm.at[pl.ds(0, CH)], bufs[b], sems[b]).wait()

        _gather(0, 0)

        def body(jj, carry):
            j0 = jj * 2
            _gather(j0 + 1, 1)
            _drain(0)
            pltpu.sync_copy(bufs[0], acc.at[didx0], add=True)

            @pl.when(jj < NCHUNK // 2 - 1)
            def _next():
                _gather(j0 + 2, 0)
            _drain(1)
            pltpu.sync_copy(bufs[1], acc.at[didx1], add=True)
            return carry
        lax.fori_loop(0, NCHUNK // 2, body, 0)
        if NCHUNK % 2:
            _gather(NCHUNK - 1, 0)
            _drain(0)
            pltpu.sync_copy(bufs[0], acc.at[didx0], add=True)
        plsc.subcore_barrier()

        off = pl.multiple_of(s * SROWS, 8)
        ooff = pl.multiple_of(c * N + s * SROWS, 8)

        @pl.when(s < NS - 1)
        def _wb():
            pltpu.sync_copy(acc.at[pl.ds(off, SROWS)],
                            out_hbm.at[pl.ds(ooff, SROWS)])

        @pl.when(s == NS - 1)
        def _wb_last():
            pltpu.sync_copy(acc.at[pl.ds(off, SROWS_LAST)],
                            out_hbm.at[pl.ds(ooff, SROWS_LAST)])

    return agg_kernel


def _make_sc_deg(interpret=False):
    @functools.partial(
        pl.kernel,
        out_type=jax.ShapeDtypeStruct((NW * N,), jnp.float32),
        mesh=_mesh(),
        compiler_params=pltpu.CompilerParams(needs_layout_passes=False),
        scratch_types=[
            pltpu.VMEM((EPW,), jnp.int32),
            pltpu.VMEM((N,), jnp.float32),
        ],
        interpret=interpret,
    )
    def deg_kernel(dst_hbm, out_hbm, didx, dloc):
        c = lax.axis_index("c")
        s = lax.axis_index("s")
        wid = c * NS + s

        def z(i, carry):
            dloc[pl.ds(i * 16, 16)] = jnp.zeros((16,), jnp.float32)
            return carry
        lax.fori_loop(0, N // 16, z, 0)

        pltpu.sync_copy(
            dst_hbm.at[pl.ds(pl.multiple_of(wid * EPW, 8), EPW)], didx)
        ones = jnp.ones((16,), jnp.float32)
        lanes = lax.iota(jnp.int32, 16)

        def body(i, carry):
            idx = didx[pl.ds(i * 16, 16)]
            for l in range(16):
                plsc.addupdate_scatter(dloc, [idx], ones, mask=lanes == l)
            return carry
        lax.fori_loop(0, EPW // 16, body, 0)

        pltpu.sync_copy(dloc, out_hbm.at[pl.ds(
            pl.multiple_of(wid * N, 8), N)])

    return deg_kernel


def _make_sc_pool(interpret=False):
    @functools.partial(
        pl.kernel,
        out_type=jax.ShapeDtypeStruct((B * KP, H), jnp.float32),
        mesh=_mesh(),
        compiler_params=pltpu.CompilerParams(needs_layout_passes=False),
        scratch_types=[
            pltpu.VMEM((N + 16,), jnp.float32),
            pltpu.VMEM((N,), jnp.int32),
            pltpu.VMEM((K,), jnp.int32),
            pltpu.VMEM((KP, H), jnp.float32),
            pltpu.SemaphoreType.DMA,
        ],
        interpret=interpret,
    )
    def pool_kernel(h_hbm, keys_hbm, batch_hbm, out_hbm,
                    keys_v, batch_v, sel_v, rows_v, sem):
        c = lax.axis_index("c")
        s = lax.axis_index("s")
        wid = c * NS + s
        pltpu.sync_copy(keys_hbm, keys_v.at[pl.ds(0, N)])
        pltpu.sync_copy(batch_hbm, batch_v)

        for ki in range(K, KP):
            for l in range(H // 16):
                rows_v[ki, pl.ds(l * 16, 16)] = jnp.zeros((16,), jnp.float32)

        for gi in range(GPW):
            g = wid * GPW + gi

            def cbody(i, carry):
                st, ct = carry
                bv = batch_v[pl.ds(i * 16, 16)]
                st = st + jnp.sum(jnp.where(bv < g, 1, 0))
                ct = ct + jnp.sum(jnp.where(bv == g, 1, 0))
                return st, ct
            start, cnt = lax.fori_loop(
                0, NKC, cbody, (jnp.int32(0), jnp.int32(0)))

            c0 = start // 16
            c1 = (start + cnt + 15) // 16

            def select(ki, carry):
                def scan(ci, sc_carry):
                    bk, bp = sc_carry
                    off = ci * 16
                    kv = keys_v[pl.ds(off, 16)]
                    pos = off + lax.iota(jnp.int32, 16)
                    valid = (pos >= start) & (pos < start + cnt)
                    kv = jnp.where(valid, kv, -1.0)
                    take = kv > bk
                    return (jnp.where(take, kv, bk),
                            jnp.where(take, pos, bp))
                bk, bp = lax.fori_loop(
                    c0, c1, scan,
                    (jnp.full((16,), -1.0, jnp.float32),
                     jnp.full((16,), N, jnp.int32)))
                m = jnp.max(bk)
                p = jnp.min(jnp.where(bk >= m, bp, N))
                plsc.store_scatter(
                    keys_v, [jnp.full((16,), p, jnp.int32)],
                    jnp.full((16,), -1.0, jnp.float32))
                plsc.store_scatter(
                    sel_v, [jnp.full((16,), ki, jnp.int32)],
                    jnp.full((16,), jnp.minimum(p, N - 1), jnp.int32))
                return carry
            lax.fori_loop(0, K, select, 0)

            pltpu.async_copy(h_hbm.at[sel_v], rows_v.at[pl.ds(0, K)],
                             sem).wait()

            for ki in range(K):
                @pl.when(cnt <= ki)
                def _zero(ki=ki):
                    for l in range(H // 16):
                        rows_v[ki, pl.ds(l * 16, 16)] = (
                            jnp.zeros((16,), jnp.float32))

            pltpu.sync_copy(rows_v, out_hbm.at[pl.ds(
                pl.multiple_of(g * KP, 8), KP)])

    return pool_kernel


def _tc_layer(a0, a1, degt, h, Wl, bl2, Wr, interpret=False):
    R = 1000

    def body(a0_r, a1_r, d_r, h_r, wl_r, bl_r, wr_r, o_r):
        dsum = jnp.sum(d_r[...], axis=1, keepdims=True)
        rdeg = 1.0 / jnp.maximum(dsum, 1.0)
        agg = (a0_r[...] + a1_r[...]) * rdeg
        o_r[...] = jnp.maximum(
            jnp.dot(agg, wl_r[...], preferred_element_type=jnp.float32)
            + jnp.dot(h_r[...], wr_r[...], preferred_element_type=jnp.float32)
            + bl_r[...], 0.0)

    return pl.pallas_call(
        body,
        grid=(N // R,),
        in_specs=[
            pl.BlockSpec((R, H), lambda i: (i, 0)),
            pl.BlockSpec((R, H), lambda i: (i, 0)),
            pl.BlockSpec((R, NW), lambda i: (i, 0)),
            pl.BlockSpec((R, H), lambda i: (i, 0)),
            pl.BlockSpec((H, H), lambda i: (0, 0)),
            pl.BlockSpec((1, H), lambda i: (0, 0)),
            pl.BlockSpec((H, H), lambda i: (0, 0)),
        ],
        out_specs=pl.BlockSpec((R, H), lambda i: (i, 0)),
        out_shape=jax.ShapeDtypeStruct((N, H), jnp.float32),
        interpret=interpret,
    )(a0, a1, degt, h, Wl, bl2, Wr)


def _tc_head(pooledT, Wck2, bc2, W1e, b12, W2, b22, interpret=False):
    def body(x0, x1, x2, x3, x4, wck_r, bc_r, w1e_r, b1_r, w2_r, b2_r,
             out_r, z1_s):
        i = pl.program_id(0)
        xs = (x0, x1, x2, x3, x4)
        ct = jnp.zeros((B, CONV_OUT), jnp.float32)
        for kw in range(KW):
            ct = ct + jnp.dot(xs[kw][...], wck_r[pl.ds(kw * H, H), :],
                              preferred_element_type=jnp.float32)
        crelu = jnp.maximum(ct + bc_r[...], 0.0)
        contrib = jnp.dot(crelu, w1e_r[...].reshape(CONV_OUT, H),
                          preferred_element_type=jnp.float32)

        @pl.when(i == 0)
        def _init():
            z1_s[...] = contrib

        @pl.when(i > 0)
        def _acc():
            z1_s[...] = z1_s[...] + contrib

        @pl.when(i == NT - 1)
        def _finish():
            z1 = jnp.maximum(z1_s[...] + b1_r[...], 0.0)
            z2 = jnp.dot(z1, w2_r[...],
                         preferred_element_type=jnp.float32) + b2_r[...]
            m = jnp.max(z2, axis=1, keepdims=True)
            lse = jnp.log(jnp.sum(jnp.exp(z2 - m), axis=1, keepdims=True))
            out_r[...] = z2 - m - lse

    return pl.pallas_call(
        body,
        grid=(NT,),
        in_specs=[
            pl.BlockSpec((B, H), lambda i: (i, 0)),
            pl.BlockSpec((B, H), lambda i: (i + 1, 0)),
            pl.BlockSpec((B, H), lambda i: (i + 2, 0)),
            pl.BlockSpec((B, H), lambda i: (i + 3, 0)),
            pl.BlockSpec((B, H), lambda i: (i + 4, 0)),
            pl.BlockSpec((KW * H, CONV_OUT), lambda i: (0, 0)),
            pl.BlockSpec((1, CONV_OUT), lambda i: (0, 0)),
            pl.BlockSpec((1, CONV_OUT, H), lambda i: (i, 0, 0)),
            pl.BlockSpec((1, H), lambda i: (0, 0)),
            pl.BlockSpec((H, C), lambda i: (0, 0)),
            pl.BlockSpec((1, C), lambda i: (0, 0)),
        ],
        out_specs=pl.BlockSpec((B, C), lambda i: (0, 0)),
        out_shape=jax.ShapeDtypeStruct((B, C), jnp.float32),
        scratch_shapes=[pltpu.VMEM((B, H), jnp.float32)],
        interpret=interpret,
    )(pooledT, pooledT, pooledT, pooledT, pooledT,
      Wck2, bc2, W1e, b12, W2, b22)


_sc_agg = None
_sc_deg = None
_sc_pool = None


def _get_sc_kernels():
    global _sc_agg, _sc_deg, _sc_pool
    if _sc_agg is None:
        _sc_agg = _make_sc_agg(H)
        _sc_deg = _make_sc_deg()
        _sc_pool = _make_sc_pool()
    return _sc_agg, _sc_deg, _sc_pool


def kernel(x, edge_index, batch, Wl1, bl1, Wr1, Wl2, bl2, Wr2,
           Wl3, bl3, Wr3, Wc, bc, W1, b1, W2, b2):
    agg, deg, pool = _get_sc_kernels()
    src = edge_index[0]
    dst = edge_index[1]

    degt = deg(dst).reshape(NW, N).T

    p = agg(x, src, dst)
    h = _tc_layer(p[:N], p[N:], degt, x, Wl1, bl1[None, :], Wr1)
    p = agg(h, src, dst)
    h = _tc_layer(p[:N], p[N:], degt, h, Wl2, bl2[None, :], Wr2)
    p = agg(h, src, dst)
    h = _tc_layer(p[:N], p[N:], degt, h, Wl3, bl3[None, :], Wr3)

    keys = h[:, H - 1]
    pooled = pool(h, keys, batch)

    pooledT = (pooled.reshape(B, KP, H)[:, :K, :]
               .transpose(1, 0, 2).reshape(K * B, H))
    Wck2 = Wc.transpose(2, 1, 0).reshape(KW * H, CONV_OUT)
    W1e = W1.reshape(CONV_OUT, NT, H).transpose(1, 0, 2)
    return _tc_head(pooledT, Wck2, bc[None, :], W1e, b1[None, :],
                    W2, b2[None, :])


# speedup vs baseline: 9.6931x; 1.3760x over previous
import functools

import jax
import jax.numpy as jnp
from jax import lax
from jax.experimental import pallas as pl
from jax.experimental.pallas import tpu as pltpu
from jax.experimental.pallas import tpu_sc as plsc

N = 10000
E = 320000
H = 128
B = 64
K = 30
C = 10
CONV_OUT = 32
KW = 5
NT = K - KW + 1

NC = 2
NS = 16
NW = NC * NS
EPW = E // NW
CH = 80
NCHUNK = EPW // CH
SROWS = 624
SROWS_LAST = N - (NS - 1) * SROWS
ZR = 16
KP = 32
GPW = B // NW
NKC = N // 16

def _mesh():
    return plsc.VectorSubcoreMesh(core_axis_name="c", subcore_axis_name="s",
                                  num_cores=NC, num_subcores=NS)


def _make_sc_agg(W, interpret=False):
    @functools.partial(
        pl.kernel,
        out_type=jax.ShapeDtypeStruct((NC * N, W), jnp.float32),
        mesh=_mesh(),
        compiler_params=pltpu.CompilerParams(needs_layout_passes=False),
        scratch_types=[
            pltpu.VMEM((EPW,), jnp.int32),
            pltpu.VMEM((NCHUNK, CH), jnp.int32),
            pltpu.VMEM((CH, W), jnp.float32),
            pltpu.VMEM((CH, W), jnp.float32),
            pltpu.VMEM((ZR, W), jnp.float32),
            pltpu.VMEM_SHARED((N, W), jnp.float32),
            pltpu.SemaphoreType.DMA,
            pltpu.SemaphoreType.DMA,
        ],
        interpret=interpret,
    )
    def agg_kernel(table_hbm, src_hbm, dst3_hbm, out_hbm,
                   sidxf, didx2, rows0, rows1, zbuf, acc, sem0, sem1):
        c = lax.axis_index("c")
        s = lax.axis_index("s")
        wid = c * NS + s

        def zrow(r, carry):
            for l in range(W // 16):
                zbuf[r, pl.ds(l * 16, 16)] = jnp.zeros((16,), jnp.float32)
            return carry
        lax.fori_loop(0, ZR, zrow, 0)
        nz = jnp.where(s == NS - 1, SROWS_LAST // ZR, SROWS // ZR)

        def zslice(j, carry):
            off = pl.multiple_of(s * SROWS + j * ZR, 8)
            pltpu.sync_copy(zbuf, acc.at[pl.ds(off, ZR)])
            return carry
        lax.fori_loop(0, nz, zslice, 0)
        plsc.subcore_barrier()

        base0 = wid * EPW
        pltpu.sync_copy(
            src_hbm.at[pl.ds(pl.multiple_of(base0, 8), EPW)], sidxf)
        pltpu.sync_copy(dst3_hbm.at[wid], didx2)
        bufs = (rows0, rows1)
        sems = (sem0, sem1)

        def _gather(j, b):
            pltpu.async_copy(
                table_hbm.at[sidxf.at[pl.ds(j * CH, CH)]], bufs[b], sems[b])

        def _drain(b):
            pltpu.make_async_copy(
                table_hbm.at[pl.ds(0, CH)], bufs[b], sems[b]).wait()

        _gather(0, 0)

        def body(jj, carry):
            j0 = jj * 2
            _gather(j0 + 1, 1)
            _drain(0)
            pltpu.sync_copy(bufs[0], acc.at[didx2.at[j0]], add=True)

            @pl.when(jj < NCHUNK // 2 - 1)
            def _next():
                _gather(j0 + 2, 0)
            _drain(1)
            pltpu.sync_copy(bufs[1], acc.at[didx2.at[j0 + 1]], add=True)
            return carry
        lax.fori_loop(0, NCHUNK // 2, body, 0)
        if NCHUNK % 2:
            _gather(NCHUNK - 1, 0)
            _drain(0)
            pltpu.sync_copy(bufs[0], acc.at[didx2.at[NCHUNK - 1]], add=True)
        plsc.subcore_barrier()

        off = pl.multiple_of(s * SROWS, 8)
        ooff = pl.multiple_of(c * N + s * SROWS, 8)

        @pl.when(s < NS - 1)
        def _wb():
            pltpu.sync_copy(acc.at[pl.ds(off, SROWS)],
                            out_hbm.at[pl.ds(ooff, SROWS)])

        @pl.when(s == NS - 1)
        def _wb_last():
            pltpu.sync_copy(acc.at[pl.ds(off, SROWS_LAST)],
                            out_hbm.at[pl.ds(ooff, SROWS_LAST)])

    return agg_kernel


def _make_sc_deg(interpret=False):
    @functools.partial(
        pl.kernel,
        out_type=jax.ShapeDtypeStruct((NW * N,), jnp.float32),
        mesh=_mesh(),
        compiler_params=pltpu.CompilerParams(needs_layout_passes=False),
        scratch_types=[
            pltpu.VMEM((EPW,), jnp.int32),
            pltpu.VMEM((N,), jnp.float32),
        ],
        interpret=interpret,
    )
    def deg_kernel(dst_hbm, out_hbm, didx, dloc):
        c = lax.axis_index("c")
        s = lax.axis_index("s")
        wid = c * NS + s

        def z(i, carry):
            dloc[pl.ds(i * 16, 16)] = jnp.zeros((16,), jnp.float32)
            return carry
        lax.fori_loop(0, N // 16, z, 0)

        pltpu.sync_copy(
            dst_hbm.at[pl.ds(pl.multiple_of(wid * EPW, 8), EPW)], didx)
        ones = jnp.ones((16,), jnp.float32)
        lanes = lax.iota(jnp.int32, 16)

        def body(i, carry):
            idx = didx[pl.ds(i * 16, 16)]
            for l in range(16):
                plsc.addupdate_scatter(dloc, [idx], ones, mask=lanes == l)
            return carry
        lax.fori_loop(0, EPW // 16, body, 0)

        pltpu.sync_copy(dloc, out_hbm.at[pl.ds(
            pl.multiple_of(wid * N, 8), N)])

    return deg_kernel


def _make_sc_pool(interpret=False):
    @functools.partial(
        pl.kernel,
        out_type=jax.ShapeDtypeStruct((B * KP, H), jnp.float32),
        mesh=_mesh(),
        compiler_params=pltpu.CompilerParams(needs_layout_passes=False),
        scratch_types=[
            pltpu.VMEM((N + 16,), jnp.float32),
            pltpu.VMEM((N,), jnp.int32),
            pltpu.VMEM((K,), jnp.int32),
            pltpu.VMEM((KP, H), jnp.float32),
            pltpu.SemaphoreType.DMA,
        ],
        interpret=interpret,
    )
    def pool_kernel(h_hbm, keys_hbm, batch_hbm, out_hbm,
                    keys_v, batch_v, sel_v, rows_v, sem):
        c = lax.axis_index("c")
        s = lax.axis_index("s")
        wid = c * NS + s
        pltpu.sync_copy(keys_hbm, keys_v.at[pl.ds(0, N)])
        pltpu.sync_copy(batch_hbm, batch_v)

        for ki in range(K, KP):
            for l in range(H // 16):
                rows_v[ki, pl.ds(l * 16, 16)] = jnp.zeros((16,), jnp.float32)

        for gi in range(GPW):
            g = wid * GPW + gi

            def cbody(i, carry):
                st, ct = carry
                bv = batch_v[pl.ds(i * 16, 16)]
                st = st + jnp.sum(jnp.where(bv < g, 1, 0))
                ct = ct + jnp.sum(jnp.where(bv == g, 1, 0))
                return st, ct
            start, cnt = lax.fori_loop(
                0, NKC, cbody, (jnp.int32(0), jnp.int32(0)))

            c0 = start // 16
            c1 = (start + cnt + 15) // 16

            def select(ki, carry):
                def scan(ci, sc_carry):
                    bk, bp = sc_carry
                    off = ci * 16
                    kv = keys_v[pl.ds(off, 16)]
                    pos = off + lax.iota(jnp.int32, 16)
                    valid = (pos >= start) & (pos < start + cnt)
                    kv = jnp.where(valid, kv, -1.0)
                    take = kv > bk
                    return (jnp.where(take, kv, bk),
                            jnp.where(take, pos, bp))
                bk, bp = lax.fori_loop(
                    c0, c1, scan,
                    (jnp.full((16,), -1.0, jnp.float32),
                     jnp.full((16,), N, jnp.int32)))
                m = jnp.max(bk)
                p = jnp.min(jnp.where(bk >= m, bp, N))
                plsc.store_scatter(
                    keys_v, [jnp.full((16,), p, jnp.int32)],
                    jnp.full((16,), -1.0, jnp.float32))
                plsc.store_scatter(
                    sel_v, [jnp.full((16,), ki, jnp.int32)],
                    jnp.full((16,), jnp.minimum(p, N - 1), jnp.int32))
                return carry
            lax.fori_loop(0, K, select, 0)

            pltpu.async_copy(h_hbm.at[sel_v], rows_v.at[pl.ds(0, K)],
                             sem).wait()

            for ki in range(K):
                @pl.when(cnt <= ki)
                def _zero(ki=ki):
                    for l in range(H // 16):
                        rows_v[ki, pl.ds(l * 16, 16)] = (
                            jnp.zeros((16,), jnp.float32))

            pltpu.sync_copy(rows_v, out_hbm.at[pl.ds(
                pl.multiple_of(g * KP, 8), KP)])

    return pool_kernel


def _tc_layer(a0, a1, degt, h, Wl, bl2, Wr, interpret=False):
    R = 1000

    def body(a0_r, a1_r, d_r, h_r, wl_r, bl_r, wr_r, o_r):
        dsum = jnp.sum(d_r[...], axis=1, keepdims=True)
        rdeg = 1.0 / jnp.maximum(dsum, 1.0)
        agg = (a0_r[...] + a1_r[...]) * rdeg
        o_r[...] = jnp.maximum(
            jnp.dot(agg, wl_r[...], preferred_element_type=jnp.float32)
            + jnp.dot(h_r[...], wr_r[...], preferred_element_type=jnp.float32)
            + bl_r[...], 0.0)

    return pl.pallas_call(
        body,
        grid=(N // R,),
        in_specs=[
            pl.BlockSpec((R, H), lambda i: (i, 0)),
            pl.BlockSpec((R, H), lambda i: (i, 0)),
            pl.BlockSpec((R, NW), lambda i: (i, 0)),
            pl.BlockSpec((R, H), lambda i: (i, 0)),
            pl.BlockSpec((H, H), lambda i: (0, 0)),
            pl.BlockSpec((1, H), lambda i: (0, 0)),
            pl.BlockSpec((H, H), lambda i: (0, 0)),
        ],
        out_specs=pl.BlockSpec((R, H), lambda i: (i, 0)),
        out_shape=jax.ShapeDtypeStruct((N, H), jnp.float32),
        interpret=interpret,
    )(a0, a1, degt, h, Wl, bl2, Wr)


def _tc_head(pooledT, Wck2, bc2, W1e, b12, W2, b22, interpret=False):
    def body(x0, x1, x2, x3, x4, wck_r, bc_r, w1e_r, b1_r, w2_r, b2_r,
             out_r, z1_s):
        i = pl.program_id(0)
        xs = (x0, x1, x2, x3, x4)
        ct = jnp.zeros((B, CONV_OUT), jnp.float32)
        for kw in range(KW):
            ct = ct + jnp.dot(xs[kw][...], wck_r[pl.ds(kw * H, H), :],
                              preferred_element_type=jnp.float32)
        crelu = jnp.maximum(ct + bc_r[...], 0.0)
        contrib = jnp.dot(crelu, w1e_r[...].reshape(CONV_OUT, H),
                          preferred_element_type=jnp.float32)

        @pl.when(i == 0)
        def _init():
            z1_s[...] = contrib

        @pl.when(i > 0)
        def _acc():
            z1_s[...] = z1_s[...] + contrib

        @pl.when(i == NT - 1)
        def _finish():
            z1 = jnp.maximum(z1_s[...] + b1_r[...], 0.0)
            z2 = jnp.dot(z1, w2_r[...],
                         preferred_element_type=jnp.float32) + b2_r[...]
            m = jnp.max(z2, axis=1, keepdims=True)
            lse = jnp.log(jnp.sum(jnp.exp(z2 - m), axis=1, keepdims=True))
            out_r[...] = z2 - m - lse

    return pl.pallas_call(
        body,
        grid=(NT,),
        in_specs=[
            pl.BlockSpec((B, H), lambda i: (i, 0)),
            pl.BlockSpec((B, H), lambda i: (i + 1, 0)),
            pl.BlockSpec((B, H), lambda i: (i + 2, 0)),
            pl.BlockSpec((B, H), lambda i: (i + 3, 0)),
            pl.BlockSpec((B, H), lambda i: (i + 4, 0)),
            pl.BlockSpec((KW * H, CONV_OUT), lambda i: (0, 0)),
            pl.BlockSpec((1, CONV_OUT), lambda i: (0, 0)),
            pl.BlockSpec((1, CONV_OUT, H), lambda i: (i, 0, 0)),
            pl.BlockSpec((1, H), lambda i: (0, 0)),
            pl.BlockSpec((H, C), lambda i: (0, 0)),
            pl.BlockSpec((1, C), lambda i: (0, 0)),
        ],
        out_specs=pl.BlockSpec((B, C), lambda i: (0, 0)),
        out_shape=jax.ShapeDtypeStruct((B, C), jnp.float32),
        scratch_shapes=[pltpu.VMEM((B, H), jnp.float32)],
        interpret=interpret,
    )(pooledT, pooledT, pooledT, pooledT, pooledT,
      Wck2, bc2, W1e, b12, W2, b22)


_sc_agg = None
_sc_deg = None
_sc_pool = None


def _get_sc_kernels():
    global _sc_agg, _sc_deg, _sc_pool
    if _sc_agg is None:
        _sc_agg = _make_sc_agg(H)
        _sc_deg = _make_sc_deg()
        _sc_pool = _make_sc_pool()
    return _sc_agg, _sc_deg, _sc_pool


def kernel(x, edge_index, batch, Wl1, bl1, Wr1, Wl2, bl2, Wr2,
           Wl3, bl3, Wr3, Wc, bc, W1, b1, W2, b2):
    agg, deg, pool = _get_sc_kernels()
    src = edge_index[0]
    dst = edge_index[1]

    degt = deg(dst).reshape(NW, N).T
    dst3 = dst.reshape(NW, NCHUNK, CH)

    p = agg(x, src, dst3)
    h = _tc_layer(p[:N], p[N:], degt, x, Wl1, bl1[None, :], Wr1)
    p = agg(h, src, dst3)
    h = _tc_layer(p[:N], p[N:], degt, h, Wl2, bl2[None, :], Wr2)
    p = agg(h, src, dst3)
    h = _tc_layer(p[:N], p[N:], degt, h, Wl3, bl3[None, :], Wr3)

    keys = h[:, H - 1]
    pooled = pool(h, keys, batch)

    pooledT = (pooled.reshape(B, KP, H)[:, :K, :]
               .transpose(1, 0, 2).reshape(K * B, H))
    Wck2 = Wc.transpose(2, 1, 0).reshape(KW * H, CONV_OUT)
    W1e = W1.reshape(CONV_OUT, NT, H).transpose(1, 0, 2)
    return _tc_head(pooledT, Wck2, bc[None, :], W1e, b1[None, :],
                    W2, b2[None, :])


# speedup vs baseline: 9.7185x; 1.0026x over previous
import functools

import jax
import jax.numpy as jnp
from jax import lax
from jax.experimental import pallas as pl
from jax.experimental.pallas import tpu as pltpu
from jax.experimental.pallas import tpu_sc as plsc

N = 10000
E = 320000
H = 128
B = 64
K = 30
C = 10
CONV_OUT = 32
KW = 5
NT = K - KW + 1

NC = 2
NS = 16
NW = NC * NS
EPW = E // NW
CH = 80
NCHUNK = EPW // CH
SROWS = 624
SROWS_LAST = N - (NS - 1) * SROWS
ZR = 16
KP = 32
GPW = B // NW
NKC = N // 16

def _mesh():
    return plsc.VectorSubcoreMesh(core_axis_name="c", subcore_axis_name="s",
                                  num_cores=NC, num_subcores=NS)


def _make_sc_agg(W, interpret=False):
    @functools.partial(
        pl.kernel,
        out_type=jax.ShapeDtypeStruct((NC * N, W), jnp.float32),
        mesh=_mesh(),
        compiler_params=pltpu.CompilerParams(needs_layout_passes=False),
        scratch_types=[
            pltpu.VMEM((EPW,), jnp.int32),
            pltpu.VMEM((NCHUNK, CH), jnp.int32),
            pltpu.VMEM((CH, W), jnp.float32),
            pltpu.VMEM((CH, W), jnp.float32),
            pltpu.VMEM_SHARED((N, W), jnp.float32),
            pltpu.SemaphoreType.DMA,
            pltpu.SemaphoreType.DMA,
        ],
        interpret=interpret,
    )
    def agg_kernel(table_hbm, src_hbm, dst3_hbm, out_hbm,
                   sidxf, didx2, rows0, rows1, acc, sem0, sem1):
        c = lax.axis_index("c")
        s = lax.axis_index("s")
        wid = c * NS + s

        zbuf = rows0.at[pl.ds(0, ZR)]

        def zrow(r, carry):
            for l in range(W // 16):
                rows0[r, pl.ds(l * 16, 16)] = jnp.zeros((16,), jnp.float32)
            return carry
        lax.fori_loop(0, ZR, zrow, 0)
        nz = jnp.where(s == NS - 1, SROWS_LAST // ZR, SROWS // ZR)

        def zslice(j, carry):
            off = pl.multiple_of(s * SROWS + j * ZR, 8)
            pltpu.sync_copy(zbuf, acc.at[pl.ds(off, ZR)])
            return carry
        lax.fori_loop(0, nz, zslice, 0)
        plsc.subcore_barrier()

        base0 = wid * EPW
        pltpu.sync_copy(
            src_hbm.at[pl.ds(pl.multiple_of(base0, 8), EPW)], sidxf)
        pltpu.sync_copy(dst3_hbm.at[wid], didx2)
        bufs = (rows0, rows1)
        sems = (sem0, sem1)

        def _gather(j, b):
            pltpu.async_copy(
                table_hbm.at[sidxf.at[pl.ds(j * CH, CH)]], bufs[b], sems[b])

        def _drain(b):
            pltpu.make_async_copy(
                table_hbm.at[pl.ds(0, CH)], bufs[b], sems[b]).wait()

        NBUF = 2
        for b in range(NBUF):
            _gather(b, b)

        def body(g, carry):
            for b in range(NBUF):
                j = g * NBUF + b
                _drain(b)
                pltpu.sync_copy(bufs[b], acc.at[didx2.at[j]], add=True)

                @pl.when(j + NBUF < NCHUNK)
                def _pref(b=b, j=j):
                    _gather(j + NBUF, b)
            return carry
        lax.fori_loop(0, NCHUNK // NBUF, body, 0)
        for j in range((NCHUNK // NBUF) * NBUF, NCHUNK):
            b = j % NBUF
            _drain(b)
            pltpu.sync_copy(bufs[b], acc.at[didx2.at[j]], add=True)
        plsc.subcore_barrier()

        off = pl.multiple_of(s * SROWS, 8)
        ooff = pl.multiple_of(c * N + s * SROWS, 8)

        @pl.when(s < NS - 1)
        def _wb():
            pltpu.sync_copy(acc.at[pl.ds(off, SROWS)],
                            out_hbm.at[pl.ds(ooff, SROWS)])

        @pl.when(s == NS - 1)
        def _wb_last():
            pltpu.sync_copy(acc.at[pl.ds(off, SROWS_LAST)],
                            out_hbm.at[pl.ds(ooff, SROWS_LAST)])

    return agg_kernel


def _make_sc_deg(interpret=False):
    @functools.partial(
        pl.kernel,
        out_type=jax.ShapeDtypeStruct((NW * N,), jnp.float32),
        mesh=_mesh(),
        compiler_params=pltpu.CompilerParams(needs_layout_passes=False),
        scratch_types=[
            pltpu.VMEM((EPW,), jnp.int32),
            pltpu.VMEM((N,), jnp.float32),
        ],
        interpret=interpret,
    )
    def deg_kernel(dst_hbm, out_hbm, didx, dloc):
        c = lax.axis_index("c")
        s = lax.axis_index("s")
        wid = c * NS + s

        def z(i, carry):
            dloc[pl.ds(i * 16, 16)] = jnp.zeros((16,), jnp.float32)
            return carry
        lax.fori_loop(0, N // 16, z, 0)

        pltpu.sync_copy(
            dst_hbm.at[pl.ds(pl.multiple_of(wid * EPW, 8), EPW)], didx)
        ones = jnp.ones((16,), jnp.float32)
        lanes = lax.iota(jnp.int32, 16)

        def body(i, carry):
            idx = didx[pl.ds(i * 16, 16)]
            for l in range(16):
                plsc.addupdate_scatter(dloc, [idx], ones, mask=lanes == l)
            return carry
        lax.fori_loop(0, EPW // 16, body, 0)

        pltpu.sync_copy(dloc, out_hbm.at[pl.ds(
            pl.multiple_of(wid * N, 8), N)])

    return deg_kernel


def _make_sc_pool(interpret=False):
    @functools.partial(
        pl.kernel,
        out_type=jax.ShapeDtypeStruct((B * KP, H), jnp.float32),
        mesh=_mesh(),
        compiler_params=pltpu.CompilerParams(needs_layout_passes=False),
        scratch_types=[
            pltpu.VMEM((N + 16,), jnp.float32),
            pltpu.VMEM((N,), jnp.int32),
            pltpu.VMEM((K,), jnp.int32),
            pltpu.VMEM((KP, H), jnp.float32),
            pltpu.SemaphoreType.DMA,
        ],
        interpret=interpret,
    )
    def pool_kernel(h_hbm, keys_hbm, batch_hbm, out_hbm,
                    keys_v, batch_v, sel_v, rows_v, sem):
        c = lax.axis_index("c")
        s = lax.axis_index("s")
        wid = c * NS + s
        pltpu.sync_copy(keys_hbm, keys_v.at[pl.ds(0, N)])
        pltpu.sync_copy(batch_hbm, batch_v)

        for ki in range(K, KP):
            for l in range(H // 16):
                rows_v[ki, pl.ds(l * 16, 16)] = jnp.zeros((16,), jnp.float32)

        for gi in range(GPW):
            g = wid * GPW + gi

            def cbody(i, carry):
                st, ct = carry
                bv = batch_v[pl.ds(i * 16, 16)]
                st = st + jnp.sum(jnp.where(bv < g, 1, 0))
                ct = ct + jnp.sum(jnp.where(bv == g, 1, 0))
                return st, ct
            start, cnt = lax.fori_loop(
                0, NKC, cbody, (jnp.int32(0), jnp.int32(0)))

            c0 = start // 16
            c1 = (start + cnt + 15) // 16

            def select(ki, carry):
                def scan(ci, sc_carry):
                    bk, bp = sc_carry
                    off = ci * 16
                    kv = keys_v[pl.ds(off, 16)]
                    pos = off + lax.iota(jnp.int32, 16)
                    valid = (pos >= start) & (pos < start + cnt)
                    kv = jnp.where(valid, kv, -1.0)
                    take = kv > bk
                    return (jnp.where(take, kv, bk),
                            jnp.where(take, pos, bp))
                bk, bp = lax.fori_loop(
                    c0, c1, scan,
                    (jnp.full((16,), -1.0, jnp.float32),
                     jnp.full((16,), N, jnp.int32)))
                m = jnp.max(bk)
                p = jnp.min(jnp.where(bk >= m, bp, N))
                plsc.store_scatter(
                    keys_v, [jnp.full((16,), p, jnp.int32)],
                    jnp.full((16,), -1.0, jnp.float32))
                plsc.store_scatter(
                    sel_v, [jnp.full((16,), ki, jnp.int32)],
                    jnp.full((16,), jnp.minimum(p, N - 1), jnp.int32))
                return carry
            lax.fori_loop(0, K, select, 0)

            pltpu.async_copy(h_hbm.at[sel_v], rows_v.at[pl.ds(0, K)],
                             sem).wait()

            for ki in range(K):
                @pl.when(cnt <= ki)
                def _zero(ki=ki):
                    for l in range(H // 16):
                        rows_v[ki, pl.ds(l * 16, 16)] = (
                            jnp.zeros((16,), jnp.float32))

            pltpu.sync_copy(rows_v, out_hbm.at[pl.ds(
                pl.multiple_of(g * KP, 8), KP)])

    return pool_kernel


def _tc_layer(a0, a1, degt, h, Wl, bl2, Wr, interpret=False):
    R = 1000

    def body(a0_r, a1_r, d_r, h_r, wl_r, bl_r, wr_r, o_r):
        dsum = jnp.sum(d_r[...], axis=1, keepdims=True)
        rdeg = 1.0 / jnp.maximum(dsum, 1.0)
        agg = (a0_r[...] + a1_r[...]) * rdeg
        o_r[...] = jnp.maximum(
            jnp.dot(agg, wl_r[...], preferred_element_type=jnp.float32)
            + jnp.dot(h_r[...], wr_r[...], preferred_element_type=jnp.float32)
            + bl_r[...], 0.0)

    return pl.pallas_call(
        body,
        grid=(N // R,),
        in_specs=[
            pl.BlockSpec((R, H), lambda i: (i, 0)),
            pl.BlockSpec((R, H), lambda i: (i, 0)),
            pl.BlockSpec((R, NW), lambda i: (i, 0)),
            pl.BlockSpec((R, H), lambda i: (i, 0)),
            pl.BlockSpec((H, H), lambda i: (0, 0)),
            pl.BlockSpec((1, H), lambda i: (0, 0)),
            pl.BlockSpec((H, H), lambda i: (0, 0)),
        ],
        out_specs=pl.BlockSpec((R, H), lambda i: (i, 0)),
        out_shape=jax.ShapeDtypeStruct((N, H), jnp.float32),
        interpret=interpret,
    )(a0, a1, degt, h, Wl, bl2, Wr)


def _tc_head(pooledT, Wck2, bc2, W1e, b12, W2, b22, interpret=False):
    def body(x0, x1, x2, x3, x4, wck_r, bc_r, w1e_r, b1_r, w2_r, b2_r,
             out_r, z1_s):
        i = pl.program_id(0)
        xs = (x0, x1, x2, x3, x4)
        ct = jnp.zeros((B, CONV_OUT), jnp.float32)
        for kw in range(KW):
            ct = ct + jnp.dot(xs[kw][...], wck_r[pl.ds(kw * H, H), :],
                              preferred_element_type=jnp.float32)
        crelu = jnp.maximum(ct + bc_r[...], 0.0)
        contrib = jnp.dot(crelu, w1e_r[...].reshape(CONV_OUT, H),
                          preferred_element_type=jnp.float32)

        @pl.when(i == 0)
        def _init():
            z1_s[...] = contrib

        @pl.when(i > 0)
        def _acc():
            z1_s[...] = z1_s[...] + contrib

        @pl.when(i == NT - 1)
        def _finish():
            z1 = jnp.maximum(z1_s[...] + b1_r[...], 0.0)
            z2 = jnp.dot(z1, w2_r[...],
                         preferred_element_type=jnp.float32) + b2_r[...]
            m = jnp.max(z2, axis=1, keepdims=True)
            lse = jnp.log(jnp.sum(jnp.exp(z2 - m), axis=1, keepdims=True))
            out_r[...] = z2 - m - lse

    return pl.pallas_call(
        body,
        grid=(NT,),
        in_specs=[
            pl.BlockSpec((B, H), lambda i: (i, 0)),
            pl.BlockSpec((B, H), lambda i: (i + 1, 0)),
            pl.BlockSpec((B, H), lambda i: (i + 2, 0)),
            pl.BlockSpec((B, H), lambda i: (i + 3, 0)),
            pl.BlockSpec((B, H), lambda i: (i + 4, 0)),
            pl.BlockSpec((KW * H, CONV_OUT), lambda i: (0, 0)),
            pl.BlockSpec((1, CONV_OUT), lambda i: (0, 0)),
            pl.BlockSpec((1, CONV_OUT, H), lambda i: (i, 0, 0)),
            pl.BlockSpec((1, H), lambda i: (0, 0)),
            pl.BlockSpec((H, C), lambda i: (0, 0)),
            pl.BlockSpec((1, C), lambda i: (0, 0)),
        ],
        out_specs=pl.BlockSpec((B, C), lambda i: (0, 0)),
        out_shape=jax.ShapeDtypeStruct((B, C), jnp.float32),
        scratch_shapes=[pltpu.VMEM((B, H), jnp.float32)],
        interpret=interpret,
    )(pooledT, pooledT, pooledT, pooledT, pooledT,
      Wck2, bc2, W1e, b12, W2, b22)


_sc_agg = None
_sc_deg = None
_sc_pool = None


def _get_sc_kernels():
    global _sc_agg, _sc_deg, _sc_pool
    if _sc_agg is None:
        _sc_agg = _make_sc_agg(H)
        _sc_deg = _make_sc_deg()
        _sc_pool = _make_sc_pool()
    return _sc_agg, _sc_deg, _sc_pool


def kernel(x, edge_index, batch, Wl1, bl1, Wr1, Wl2, bl2, Wr2,
           Wl3, bl3, Wr3, Wc, bc, W1, b1, W2, b2):
    agg, deg, pool = _get_sc_kernels()
    src = edge_index[0]
    dst = edge_index[1]

    degt = deg(dst).reshape(NW, N).T
    dst3 = dst.reshape(NW, NCHUNK, CH)

    p = agg(x, src, dst3)
    h = _tc_layer(p[:N], p[N:], degt, x, Wl1, bl1[None, :], Wr1)
    p = agg(h, src, dst3)
    h = _tc_layer(p[:N], p[N:], degt, h, Wl2, bl2[None, :], Wr2)
    p = agg(h, src, dst3)
    h = _tc_layer(p[:N], p[N:], degt, h, Wl3, bl3[None, :], Wr3)

    keys = h[:, H - 1]
    pooled = pool(h, keys, batch)

    pooledT = (pooled.reshape(B, KP, H)[:, :K, :]
               .transpose(1, 0, 2).reshape(K * B, H))
    Wck2 = Wc.transpose(2, 1, 0).reshape(KW * H, CONV_OUT)
    W1e = W1.reshape(CONV_OUT, NT, H).transpose(1, 0, 2)
    return _tc_head(pooledT, Wck2, bc[None, :], W1e, b1[None, :],
                    W2, b2[None, :])
